# baseline (speedup 1.0000x reference)
import jax
import jax.numpy as jnp
from jax.experimental import pallas as pl
from jax.experimental.pallas import tpu as pltpu

_I = 3
_J = 700
_ROWS = 16384
_COLS = 1024
_BR = 1024
_N = _ROWS // _BR
_NBUF = 6


def _body(theta_ref, x_ref, o_ref, bufs, in_sems, out_sems):
    t = theta_ref[0]
    c = jnp.cos(2.0 * t)
    s = jnp.sin(2.0 * t)

    def in_cp(i):
        return pltpu.make_async_copy(
            x_ref.at[pl.ds(i * _BR, _BR), :], bufs.at[i % _NBUF], in_sems.at[i])

    def out_cp(i):
        return pltpu.make_async_copy(
            bufs.at[i % _NBUF], o_ref.at[pl.ds(i * _BR, _BR), :], out_sems.at[i])

    for i in range(_NBUF):
        in_cp(i).start()
    for i in range(_N):
        b = i % _NBUF
        in_cp(i).wait()
        xi = bufs[b, :, _I:_I + 1]
        xj = bufs[b, :, _J:_J + 1]
        bufs[b, :, _I:_I + 1] = xi * c + xj * s
        bufs[b, :, _J:_J + 1] = xi * s - xj * c
        out_cp(i).start()
        k = i + _NBUF
        if k < _N:
            out_cp(i).wait()
            in_cp(k).start()
    for i in range(_N - _NBUF, _N):
        out_cp(i).wait()


def kernel(x, theta):
    theta_arr = jnp.reshape(theta, (1,)).astype(jnp.float32)
    return pl.pallas_call(
        _body,
        in_specs=[
            pl.BlockSpec(memory_space=pltpu.SMEM),
            pl.BlockSpec(memory_space=pl.ANY),
        ],
        out_specs=pl.BlockSpec(memory_space=pl.ANY),
        out_shape=jax.ShapeDtypeStruct((_ROWS, _COLS), jnp.float32),
        scratch_shapes=[
            pltpu.VMEM((_NBUF, _BR, _COLS), jnp.float32),
            pltpu.SemaphoreType.DMA((_N,)),
            pltpu.SemaphoreType.DMA((_N,)),
        ],
    )(theta_arr, x)


# speedup vs baseline: 1.0625x; 1.0625x over previous
import jax
import jax.numpy as jnp
from jax.experimental import pallas as pl
from jax.experimental.pallas import tpu as pltpu

_I = 3
_J = 700
_ROWS = 16384
_COLS = 1024
_BR = 2048
_N = _ROWS // _BR
_NBUF = 4


def _body(theta_ref, x_ref, o_ref, bufs, in_sems, out_sems):
    t = theta_ref[0]
    c = jnp.cos(2.0 * t)
    s = jnp.sin(2.0 * t)

    def in_cp(i):
        return pltpu.make_async_copy(
            x_ref.at[pl.ds(i * _BR, _BR), :], bufs.at[i % _NBUF], in_sems.at[i])

    def out_cp(i):
        return pltpu.make_async_copy(
            bufs.at[i % _NBUF], o_ref.at[pl.ds(i * _BR, _BR), :], out_sems.at[i])

    for i in range(_NBUF):
        in_cp(i).start()
    for i in range(_N):
        b = i % _NBUF
        in_cp(i).wait()
        xi = bufs[b, :, _I:_I + 1]
        xj = bufs[b, :, _J:_J + 1]
        bufs[b, :, _I:_I + 1] = xi * c + xj * s
        bufs[b, :, _J:_J + 1] = xi * s - xj * c
        out_cp(i).start()
        k = i + _NBUF
        if k < _N:
            out_cp(i).wait()
            in_cp(k).start()
    for i in range(_N - _NBUF, _N):
        out_cp(i).wait()


def kernel(x, theta):
    theta_arr = jnp.reshape(theta, (1,)).astype(jnp.float32)
    return pl.pallas_call(
        _body,
        in_specs=[
            pl.BlockSpec(memory_space=pltpu.SMEM),
            pl.BlockSpec(memory_space=pl.ANY),
        ],
        out_specs=pl.BlockSpec(memory_space=pl.ANY),
        out_shape=jax.ShapeDtypeStruct((_ROWS, _COLS), jnp.float32),
        scratch_shapes=[
            pltpu.VMEM((_NBUF, _BR, _COLS), jnp.float32),
            pltpu.SemaphoreType.DMA((_N,)),
            pltpu.SemaphoreType.DMA((_N,)),
        ],
    )(theta_arr, x)


# speedup vs baseline: 1.0734x; 1.0102x over previous
import jax
import jax.numpy as jnp
from jax.experimental import pallas as pl
from jax.experimental.pallas import tpu as pltpu

_I = 3
_J = 700
_ROWS = 16384
_COLS = 1024
_BR = 4096
_N = _ROWS // _BR
_NBUF = 3


def _body(theta_ref, x_ref, o_ref, bufs, in_sems, out_sems):
    t = theta_ref[0]
    c = jnp.cos(2.0 * t)
    s = jnp.sin(2.0 * t)

    def in_cp(i):
        return pltpu.make_async_copy(
            x_ref.at[pl.ds(i * _BR, _BR), :], bufs.at[i % _NBUF], in_sems.at[i])

    def out_cp(i):
        return pltpu.make_async_copy(
            bufs.at[i % _NBUF], o_ref.at[pl.ds(i * _BR, _BR), :], out_sems.at[i])

    for i in range(_NBUF):
        in_cp(i).start()
    for i in range(_N):
        b = i % _NBUF
        in_cp(i).wait()
        xi = bufs[b, :, _I:_I + 1]
        xj = bufs[b, :, _J:_J + 1]
        bufs[b, :, _I:_I + 1] = xi * c + xj * s
        bufs[b, :, _J:_J + 1] = xi * s - xj * c
        out_cp(i).start()
        k = i + _NBUF
        if k < _N:
            out_cp(i).wait()
            in_cp(k).start()
    for i in range(_N - _NBUF, _N):
        out_cp(i).wait()


def kernel(x, theta):
    theta_arr = jnp.reshape(theta, (1,)).astype(jnp.float32)
    return pl.pallas_call(
        _body,
        in_specs=[
            pl.BlockSpec(memory_space=pltpu.SMEM),
            pl.BlockSpec(memory_space=pl.ANY),
        ],
        out_specs=pl.BlockSpec(memory_space=pl.ANY),
        out_shape=jax.ShapeDtypeStruct((_ROWS, _COLS), jnp.float32),
        scratch_shapes=[
            pltpu.VMEM((_NBUF, _BR, _COLS), jnp.float32),
            pltpu.SemaphoreType.DMA((_N,)),
            pltpu.SemaphoreType.DMA((_N,)),
        ],
    )(theta_arr, x)
